# Initial kernel scaffold; baseline (speedup 1.0000x reference)
import jax
import jax.numpy as jnp
from jax.experimental import pallas as pl


def kernel(x, y, F0, F1, F2, F3, F4):
    raise NotImplementedError("write your pallas kernel here")


# speedup vs baseline: 587.8845x; 587.8845x over previous
import jax
import jax.numpy as jnp
import numpy as np
from jax import lax
from jax.experimental import pallas as pl
from jax.experimental.pallas import tpu as pltpu
from jax.experimental.pallas import tpu_sc as plsc

N_PTS = 1048576
N_LVL_FEATS = 40
B = 128
SMALL_RES = (16, 32, 64)
SMALL_BASE = (0, 2048, 10240)
ST_SIZE = 43008
BIG_RES = (128, 256)


def _iota16():
    return lax.iota(jnp.int32, 16)


def _corner_setup(xs, ys, r):
    half = np.float32(0.5 * (r - 1))
    ix = (xs + 1.0) * half
    iy = (ys + 1.0) * half
    ix0 = ix.astype(jnp.int32)
    iy0 = iy.astype(jnp.int32)
    fx = ix - ix0.astype(jnp.float32)
    fy = iy - iy0.astype(jnp.float32)
    ix1 = jnp.minimum(ix0 + 1, r - 1)
    iy1 = jnp.minimum(iy0 + 1, r - 1)
    wx0 = 1.0 - fx
    wy0 = 1.0 - fy
    return ix0, iy0, ix1, iy1, wx0 * wy0, fx * wy0, wx0 * fy, fx * fy


def _prep_body(f0, f1, f2, f3, f4, st_out, t3_out, t4_out,
               buf_s, sum_s, stage4, ssub, tbuf):
    nc = 2
    wid = lax.axis_index("s") * nc + lax.axis_index("c")

    for l, r in enumerate(SMALL_RES):
        fref = (f0, f1, f2)[l]
        rr = r * r
        n = rr // 4
        f_idx = wid // 4
        off = (wid % 4) * n
        for cc in range(4):
            pltpu.sync_copy(fref.at[cc, f_idx, pl.ds(off, n)],
                            buf_s.at[cc, pl.ds(0, n)])

        @pl.loop(0, n // 16)
        def _sum_small(k):
            o = k * 16
            v = (buf_s[0, pl.ds(o, 16)] + buf_s[1, pl.ds(o, 16)]
                 + buf_s[2, pl.ds(o, 16)] + buf_s[3, pl.ds(o, 16)])
            sum_s[pl.ds(o, 16)] = v

        pltpu.sync_copy(
            sum_s.at[pl.ds(0, n)],
            st_out.at[pl.ds(SMALL_BASE[l] + f_idx * rr + off, n)])

    for (r, fref, tout) in ((128, f3, t3_out), (256, f4, t4_out)):
        rr = r * r
        nr = rr // 32
        rbase = wid * nr
        toff = jnp.minimum(rbase + nr, rr - 16)
        for cc in range(4):
            pltpu.sync_copy(fref.at[cc, :, pl.ds(rbase, nr)],
                            stage4.at[cc, :, pl.ds(0, nr)])
            pltpu.sync_copy(fref.at[cc, :, pl.ds(toff, 16)],
                            stage4.at[cc, :, pl.ds(nr, 16)])
        for f in range(8):
            @pl.loop(0, (nr + 16) // 16)
            def _sum_big(k):
                o = k * 16
                v = (stage4[0, f, pl.ds(o, 16)] + stage4[1, f, pl.ds(o, 16)]
                     + stage4[2, f, pl.ds(o, 16)] + stage4[3, f, pl.ds(o, 16)])
                ssub[f, pl.ds(o, 16)] = v

        @pl.loop(0, nr // 16)
        def _build_t(g):
            p16 = (g * 16 + _iota16()) * 16
            for k in range(16):
                f = k & 7
                sh = k >> 3
                vals = ssub[f, pl.ds(g * 16 + sh, 16)]
                plsc.store_scatter(tbuf, [p16 + k], vals)

        pltpu.sync_copy(tbuf.at[pl.ds(0, nr * 16)],
                        tout.at[pl.ds(rbase * 16, nr * 16)])


def _main_body(xr, yr, st_hbm, t3r, t4r, outr,
               xb, yb, ia3, ib3, ia4, ib4, r3a, r3b, r4a, r4b, ob, st, sem):
    nc = 2
    wid = lax.axis_index("s") * nc + lax.axis_index("c")
    pw = N_PTS // 32
    pltpu.sync_copy(st_hbm, st)

    @pl.loop(0, pw // B)
    def _chunk(ch):
        base = wid * pw + ch * B
        pltpu.sync_copy(xr.at[pl.ds(base, B)], xb)
        pltpu.sync_copy(yr.at[pl.ds(base, B)], yb)

        @pl.loop(0, B // 16)
        def _pass_a(g):
            o = g * 16
            xs = xb[pl.ds(o, 16)] * 2.0 - 1.0
            ys = yb[pl.ds(o, 16)] * 2.0 - 1.0
            for (r, ia, ib) in ((128, ia3, ib3), (256, ia4, ib4)):
                half = np.float32(0.5 * (r - 1))
                ix0 = ((xs + 1.0) * half).astype(jnp.int32)
                iy0 = ((ys + 1.0) * half).astype(jnp.int32)
                iy1 = jnp.minimum(iy0 + 1, r - 1)
                ia[pl.ds(o, 16)] = iy0 * r + ix0
                ib[pl.ds(o, 16)] = iy1 * r + ix0

        d1 = pltpu.async_copy(t3r.at[ia3], r3a, sem)
        d2 = pltpu.async_copy(t3r.at[ib3], r3b, sem)
        d3 = pltpu.async_copy(t4r.at[ia4], r4a, sem)
        d4 = pltpu.async_copy(t4r.at[ib4], r4b, sem)

        @pl.loop(0, B // 16)
        def _pass_b(g):
            o = g * 16
            xs = xb[pl.ds(o, 16)] * 2.0 - 1.0
            ys = yb[pl.ds(o, 16)] * 2.0 - 1.0
            rb40 = (o + _iota16()) * 40
            for l, r in enumerate(SMALL_RES):
                rr = r * r
                ix0, iy0, ix1, iy1, w00, w01, w10, w11 = _corner_setup(xs, ys, r)
                b00 = iy0 * r + ix0
                dx = ix1 - ix0
                b01 = b00 + dx
                b10 = iy1 * r + ix0
                b11 = b10 + dx
                for f in range(8):
                    cf = SMALL_BASE[l] + f * rr
                    v00 = plsc.load_gather(st, [b00 + cf])
                    v01 = plsc.load_gather(st, [b01 + cf])
                    v10 = plsc.load_gather(st, [b10 + cf])
                    v11 = plsc.load_gather(st, [b11 + cf])
                    of = v00 * w00 + v01 * w01 + v10 * w10 + v11 * w11
                    plsc.store_scatter(ob, [rb40 + (l * 8 + f)], of)

        d1.wait()
        d2.wait()
        d3.wait()
        d4.wait()

        @pl.loop(0, B // 16)
        def _pass_c(g):
            o = g * 16
            xs = xb[pl.ds(o, 16)] * 2.0 - 1.0
            ys = yb[pl.ds(o, 16)] * 2.0 - 1.0
            pvec = o + _iota16()
            rb40 = pvec * 40
            for li, (r, ra_, rb_) in enumerate(((128, r3a, r3b),
                                                (256, r4a, r4b))):
                _, _, _, _, w00, w01, w10, w11 = _corner_setup(xs, ys, r)
                for f in range(8):
                    kf = jnp.full((16,), f, jnp.int32)
                    kf8 = jnp.full((16,), f + 8, jnp.int32)
                    v00 = plsc.load_gather(ra_, [pvec, kf])
                    v01 = plsc.load_gather(ra_, [pvec, kf8])
                    v10 = plsc.load_gather(rb_, [pvec, kf])
                    v11 = plsc.load_gather(rb_, [pvec, kf8])
                    of = v00 * w00 + v01 * w01 + v10 * w10 + v11 * w11
                    plsc.store_scatter(ob, [rb40 + ((3 + li) * 8 + f)], of)

        pltpu.sync_copy(ob, outr.at[pl.ds(base * 40, B * 40)])


def kernel(x, y, F0, F1, F2, F3, F4):
    f32 = jnp.float32
    mesh = plsc.VectorSubcoreMesh(core_axis_name="c", subcore_axis_name="s")
    cparams = pltpu.CompilerParams(needs_layout_passes=False,
                                   use_tc_tiling_on_sc=False)

    prep = pl.kernel(
        _prep_body,
        out_type=(
            jax.ShapeDtypeStruct((ST_SIZE,), f32),
            jax.ShapeDtypeStruct((128 * 128 * 16,), f32),
            jax.ShapeDtypeStruct((256 * 256 * 16,), f32),
        ),
        mesh=mesh,
        scratch_types=[
            pltpu.VMEM((4, 1024), f32),
            pltpu.VMEM((1024,), f32),
            pltpu.VMEM((4, 8, 2064), f32),
            pltpu.VMEM((8, 2064), f32),
            pltpu.VMEM((2048 * 16,), f32),
        ],
        compiler_params=cparams,
    )

    main = pl.kernel(
        _main_body,
        out_type=jax.ShapeDtypeStruct((N_PTS * N_LVL_FEATS,), f32),
        mesh=mesh,
        scratch_types=[
            pltpu.VMEM((B,), f32),
            pltpu.VMEM((B,), f32),
            pltpu.VMEM((B,), jnp.int32),
            pltpu.VMEM((B,), jnp.int32),
            pltpu.VMEM((B,), jnp.int32),
            pltpu.VMEM((B,), jnp.int32),
            pltpu.VMEM((B, 16), f32),
            pltpu.VMEM((B, 16), f32),
            pltpu.VMEM((B, 16), f32),
            pltpu.VMEM((B, 16), f32),
            pltpu.VMEM((B * N_LVL_FEATS,), f32),
            pltpu.VMEM((ST_SIZE,), f32),
            pltpu.SemaphoreType.DMA,
        ],
        compiler_params=cparams,
    )

    fs = [F.reshape(4, 8, r * r)
          for F, r in zip((F0, F1, F2, F3, F4), (16, 32, 64, 128, 256))]
    st, t3, t4 = prep(*fs)
    out = main(x.reshape(-1), y.reshape(-1), st,
               t3.reshape(128 * 128, 16), t4.reshape(256 * 256, 16))
    return out.reshape(N_PTS, N_LVL_FEATS)


# speedup vs baseline: 615.0370x; 1.0462x over previous
import jax
import jax.numpy as jnp
import numpy as np
from jax import lax
from jax.experimental import pallas as pl
from jax.experimental.pallas import tpu as pltpu
from jax.experimental.pallas import tpu_sc as plsc

N_PTS = 1048576
N_OUT = 40
B = 256
SMALL_RES = (16, 32, 64)
SMALL_BASE = (0, 2048, 10240)
ST_SIZE = 43008


def _iota16():
    return lax.iota(jnp.int32, 16)


def _weights(xs, ys, r):
    half = np.float32(0.5 * (r - 1))
    ix = (xs + 1.0) * half
    iy = (ys + 1.0) * half
    ix0 = ix.astype(jnp.int32)
    iy0 = iy.astype(jnp.int32)
    fx = ix - ix0.astype(jnp.float32)
    fy = iy - iy0.astype(jnp.float32)
    wx0 = 1.0 - fx
    wy0 = 1.0 - fy
    return ix0, iy0, wx0 * wy0, fx * wy0, wx0 * fy, fx * fy


def _prep_body(f0, f1, f2, f3, f4, st_out, t3_out, t4_out,
               buf_s, sum_s, stage4, ssub, tbuf):
    nc = 2
    wid = lax.axis_index("s") * nc + lax.axis_index("c")

    for l, r in enumerate(SMALL_RES):
        fref = (f0, f1, f2)[l]
        rr = r * r
        n = rr // 4
        f_idx = wid // 4
        off = (wid % 4) * n
        for cc in range(4):
            pltpu.sync_copy(fref.at[cc, f_idx, pl.ds(off, n)],
                            buf_s.at[cc, pl.ds(0, n)])

        @pl.loop(0, n // 16)
        def _sum_small(k):
            o = k * 16
            v = (buf_s[0, pl.ds(o, 16)] + buf_s[1, pl.ds(o, 16)]
                 + buf_s[2, pl.ds(o, 16)] + buf_s[3, pl.ds(o, 16)])
            sum_s[pl.ds(o, 16)] = v

        pltpu.sync_copy(
            sum_s.at[pl.ds(0, n)],
            st_out.at[pl.ds(SMALL_BASE[l] + f_idx * rr + off, n)])

    for (r, fref, tout) in ((128, f3, t3_out), (256, f4, t4_out)):
        rr = r * r
        nr = rr // 32
        rbase = wid * nr
        toff = jnp.minimum(rbase + nr, rr - 16)
        for cc in range(4):
            pltpu.sync_copy(fref.at[cc, :, pl.ds(rbase, nr)],
                            stage4.at[cc, :, pl.ds(0, nr)])
            pltpu.sync_copy(fref.at[cc, :, pl.ds(toff, 16)],
                            stage4.at[cc, :, pl.ds(nr, 16)])
        for f in range(8):
            @pl.loop(0, (nr + 16) // 16)
            def _sum_big(k):
                o = k * 16
                v = (stage4[0, f, pl.ds(o, 16)] + stage4[1, f, pl.ds(o, 16)]
                     + stage4[2, f, pl.ds(o, 16)] + stage4[3, f, pl.ds(o, 16)])
                ssub[pl.ds(f * 2064 + o, 16)] = v

        pat = (_iota16() & 7) * 2064 + (_iota16() >> 3)

        @pl.loop(0, nr)
        def _build_t(p):
            tbuf[p, :] = plsc.load_gather(ssub, [pat + p])

        pltpu.sync_copy(tbuf.at[pl.ds(0, nr), :], tout.at[pl.ds(rbase, nr), :])


def _main_body(xr, yr, st_hbm, t3r, t4r, outr,
               xb, yb, ix3, ix4, r3a, r3b, r4a, r4b, ob, st,
               sem_a, sem_b, sem_o):
    nc = 2
    wid = lax.axis_index("s") * nc + lax.axis_index("c")
    pw = N_PTS // 32
    nch = pw // B
    pltpu.sync_copy(st_hbm, st)
    sems = (sem_a, sem_b)

    def do_prefetch(ch, par):
        base = wid * pw + ch * B
        for j in range(B // 128):
            pltpu.sync_copy(xr.at[pl.ds(base + j * 128, 128)], xb.at[par, j])
            pltpu.sync_copy(yr.at[pl.ds(base + j * 128, 128)], yb.at[par, j])

        @pl.loop(0, B // 128)
        def _pass_a(j):
            @pl.loop(0, 8)
            def _grp(gg):
                o = gg * 16
                xs = xb[par, j, pl.ds(o, 16)] * 2.0 - 1.0
                ys = yb[par, j, pl.ds(o, 16)] * 2.0 - 1.0
                for (r, ix_) in ((128, ix3), (256, ix4)):
                    half = np.float32(0.5 * (r - 1))
                    ix0 = ((xs + 1.0) * half).astype(jnp.int32)
                    iy0 = ((ys + 1.0) * half).astype(jnp.int32)
                    iy1 = jnp.minimum(iy0 + 1, r - 1)
                    ix_[par, 0, j, pl.ds(o, 16)] = iy0 * r + ix0
                    ix_[par, 1, j, pl.ds(o, 16)] = iy1 * r + ix0

        for j in range(B // 128):
            s = pl.ds(j * 128, 128)
            sem = sems[par]
            pltpu.async_copy(t3r.at[ix3.at[par, 0, j]], r3a.at[par, s, :], sem)
            pltpu.async_copy(t3r.at[ix3.at[par, 1, j]], r3b.at[par, s, :], sem)
            pltpu.async_copy(t4r.at[ix4.at[par, 0, j]], r4a.at[par, s, :], sem)
            pltpu.async_copy(t4r.at[ix4.at[par, 1, j]], r4b.at[par, s, :], sem)

    def drain_rows(par):
        for rows in (r3a, r3b, r4a, r4b):
            pltpu.make_async_copy(
                t3r.at[pl.ds(0, B), :], rows.at[par], sems[par]).wait()

    def blend(ch, par):
        @pl.loop(0, B // 128)
        def _blk(j):
            @pl.loop(0, 8)
            def _grp(gg):
                o = gg * 16
                xs = xb[par, j, pl.ds(o, 16)] * 2.0 - 1.0
                ys = yb[par, j, pl.ds(o, 16)] * 2.0 - 1.0
                pvec = j * 128 + o + _iota16()
                for l, r in enumerate(SMALL_RES):
                    rr = r * r
                    ix0, iy0, w00, w01, w10, w11 = _weights(xs, ys, r)
                    ix1 = jnp.minimum(ix0 + 1, r - 1)
                    iy1 = jnp.minimum(iy0 + 1, r - 1)
                    b00 = iy0 * r + ix0
                    dx = ix1 - ix0
                    b01 = b00 + dx
                    b10 = iy1 * r + ix0
                    b11 = b10 + dx
                    for f in range(8):
                        cf = SMALL_BASE[l] + f * rr
                        v00 = plsc.load_gather(st, [b00 + cf])
                        v01 = plsc.load_gather(st, [b01 + cf])
                        v10 = plsc.load_gather(st, [b10 + cf])
                        v11 = plsc.load_gather(st, [b11 + cf])
                        of = v00 * w00 + v01 * w01 + v10 * w10 + v11 * w11
                        plsc.store_scatter(
                            ob.at[par], [pvec, jnp.full((16,), l * 8 + f,
                                                        jnp.int32)], of)
                for li, (r, ra_, rb_) in enumerate(((128, r3a, r3b),
                                                    (256, r4a, r4b))):
                    _, _, w00, w01, w10, w11 = _weights(xs, ys, r)
                    for f in range(8):
                        kf = jnp.full((16,), f, jnp.int32)
                        kf8 = jnp.full((16,), f + 8, jnp.int32)
                        v00 = plsc.load_gather(ra_.at[par], [pvec, kf])
                        v01 = plsc.load_gather(ra_.at[par], [pvec, kf8])
                        v10 = plsc.load_gather(rb_.at[par], [pvec, kf])
                        v11 = plsc.load_gather(rb_.at[par], [pvec, kf8])
                        of = v00 * w00 + v01 * w01 + v10 * w10 + v11 * w11
                        plsc.store_scatter(
                            ob.at[par], [pvec, jnp.full((16,), 24 + li * 8 + f,
                                                        jnp.int32)], of)

    do_prefetch(0, 0)

    @pl.loop(0, nch, step=2)
    def _chunk2(ch0):
        for sub in range(2):
            ch = ch0 + sub

            @pl.when(ch + 1 < nch)
            def _():
                do_prefetch(ch + 1, 1 - sub)

            drain_rows(sub)

            @pl.when(ch >= 2)
            def _():
                pltpu.make_async_copy(
                    outr.at[pl.ds(0, B), :], ob.at[sub], sem_o).wait()

            blend(ch, sub)
            base = wid * pw + ch * B
            pltpu.async_copy(ob.at[sub], outr.at[pl.ds(base, B), :], sem_o)

    for par in range(2):
        pltpu.make_async_copy(
            outr.at[pl.ds(0, B), :], ob.at[par], sem_o).wait()


def kernel(x, y, F0, F1, F2, F3, F4):
    f32 = jnp.float32
    mesh = plsc.VectorSubcoreMesh(core_axis_name="c", subcore_axis_name="s")
    cparams = pltpu.CompilerParams(needs_layout_passes=False,
                                   use_tc_tiling_on_sc=False)

    prep = pl.kernel(
        _prep_body,
        out_type=(
            jax.ShapeDtypeStruct((ST_SIZE,), f32),
            jax.ShapeDtypeStruct((128 * 128, 16), f32),
            jax.ShapeDtypeStruct((256 * 256, 16), f32),
        ),
        mesh=mesh,
        scratch_types=[
            pltpu.VMEM((4, 1024), f32),
            pltpu.VMEM((1024,), f32),
            pltpu.VMEM((4, 8, 2064), f32),
            pltpu.VMEM((8 * 2064,), f32),
            pltpu.VMEM((2048, 16), f32),
        ],
        compiler_params=cparams,
    )

    main = pl.kernel(
        _main_body,
        out_type=jax.ShapeDtypeStruct((N_PTS, N_OUT), f32),
        mesh=mesh,
        scratch_types=[
            pltpu.VMEM((2, B // 128, 128), f32),
            pltpu.VMEM((2, B // 128, 128), f32),
            pltpu.VMEM((2, 2, B // 128, 128), jnp.int32),
            pltpu.VMEM((2, 2, B // 128, 128), jnp.int32),
            pltpu.VMEM((2, B, 16), f32),
            pltpu.VMEM((2, B, 16), f32),
            pltpu.VMEM((2, B, 16), f32),
            pltpu.VMEM((2, B, 16), f32),
            pltpu.VMEM((2, B, N_OUT), f32),
            pltpu.VMEM((ST_SIZE,), f32),
            pltpu.SemaphoreType.DMA,
            pltpu.SemaphoreType.DMA,
            pltpu.SemaphoreType.DMA,
        ],
        compiler_params=cparams,
    )

    fs = [F.reshape(4, 8, r * r)
          for F, r in zip((F0, F1, F2, F3, F4), (16, 32, 64, 128, 256))]
    st, t3, t4 = prep(*fs)
    return main(x.reshape(-1), y.reshape(-1), st, t3, t4)


# speedup vs baseline: 629.2226x; 1.0231x over previous
import jax
import jax.numpy as jnp
import numpy as np
from jax import lax
from jax.experimental import pallas as pl
from jax.experimental.pallas import tpu as pltpu
from jax.experimental.pallas import tpu_sc as plsc

N_PTS = 1048576
N_OUT = 40
B = 256
SMALL_RES = (16, 32)
SMALL_BASE = (0, 2048)
ST_SIZE = 10240
NSC = 256


def _iota16():
    return lax.iota(jnp.int32, 16)


def _weights(xs, ys, r):
    half = np.float32(0.5 * (r - 1))
    ix = (xs + 1.0) * half
    iy = (ys + 1.0) * half
    ix0 = ix.astype(jnp.int32)
    iy0 = iy.astype(jnp.int32)
    fx = ix - ix0.astype(jnp.float32)
    fy = iy - iy0.astype(jnp.float32)
    wx0 = 1.0 - fx
    wy0 = 1.0 - fy
    return ix0, iy0, wx0 * wy0, fx * wy0, wx0 * fy, fx * fy


def _body(xr, yr, f0, f1, f2, f3, f4, outr, t2o, t3o, t4o, st_sh,
          xb, yb, ix2, ix3, ix4, r2a, r2b, r3a, r3b, r4a, r4b, ob, st,
          sem_a, sem_b, sem_o, sem_p):
    sid = lax.axis_index("s")
    cid = lax.axis_index("c")
    wid = sid * 2 + cid
    sems = (sem_a, sem_b)

    def phase1(buf_s, sum_s, stage4, ssub, tbuf):
        for l, r in enumerate(SMALL_RES):
            fref = (f0, f1)[l]
            rr = r * r
            n = rr // 2
            f_idx = sid // 2
            off = lax.rem(sid, 2) * n
            ds = []
            for cc in range(4):
                ds.append(pltpu.async_copy(
                    fref.at[cc, f_idx, pl.ds(off, n)],
                    buf_s.at[cc, pl.ds(0, n)], sem_p))
            for d in ds:
                d.wait()

            @pl.loop(0, n // 16)
            def _sum_small(k):
                o = k * 16
                v = (buf_s[0, pl.ds(o, 16)] + buf_s[1, pl.ds(o, 16)]
                     + buf_s[2, pl.ds(o, 16)] + buf_s[3, pl.ds(o, 16)])
                sum_s[pl.ds(o, 16)] = v

            pltpu.sync_copy(
                sum_s.at[pl.ds(0, n)],
                st_sh.at[pl.ds(SMALL_BASE[l] + f_idx * rr + off, n)])

        for (r, fref, tsh) in ((64, f2, t2o), (128, f3, t3o),
                               (256, f4, t4o)):
            rr = r * r
            rows_per_tile = rr // 16
            ns = min(NSC, rows_per_tile)
            pat = (_iota16() & 7) * (ns + 16) + (_iota16() >> 3)

            @pl.loop(0, rows_per_tile // ns)
            def _subchunk(si):
                sbase = sid * rows_per_tile + si * ns
                dbase = cid * rr + sbase
                toff = jnp.minimum(sbase + ns, rr - 16)
                ds = []
                for cc in range(4):
                    ds.append(pltpu.async_copy(
                        fref.at[cc, :, pl.ds(sbase, ns)],
                        stage4.at[cc, :, pl.ds(0, ns)], sem_p))
                    ds.append(pltpu.async_copy(
                        fref.at[cc, :, pl.ds(toff, 16)],
                        stage4.at[cc, :, pl.ds(ns, 16)], sem_p))
                for d in ds:
                    d.wait()
                for f in range(8):
                    @pl.loop(0, (ns + 16) // 16)
                    def _sum_big(k):
                        o = k * 16
                        v = (stage4[0, f, pl.ds(o, 16)]
                             + stage4[1, f, pl.ds(o, 16)]
                             + stage4[2, f, pl.ds(o, 16)]
                             + stage4[3, f, pl.ds(o, 16)])
                        ssub[pl.ds(f * (ns + 16) + o, 16)] = v

                @pl.loop(0, ns)
                def _build_t(p):
                    tbuf[p, :] = plsc.load_gather(ssub, [pat + p])

                pltpu.sync_copy(tbuf.at[pl.ds(0, ns), :],
                                tsh.at[pl.ds(dbase, ns), :])

    pl.run_scoped(
        phase1,
        pltpu.VMEM((4, 512), jnp.float32),
        pltpu.VMEM((512,), jnp.float32),
        pltpu.VMEM((4, 8, NSC + 16), jnp.float32),
        pltpu.VMEM((8 * (NSC + 16),), jnp.float32),
        pltpu.VMEM((NSC, 16), jnp.float32),
    )

    plsc.subcore_barrier()

    def phase2():
        pw = N_PTS // 32
        nch = pw // B
        pltpu.sync_copy(st_sh, st)
        stream_cfg = ((64, ix2, r2a, r2b, t2o),
                      (128, ix3, r3a, r3b, t3o),
                      (256, ix4, r4a, r4b, t4o))

        def do_prefetch(ch, par):
            base = wid * pw + ch * B
            ds = []
            for j in range(B // 128):
                ds.append(pltpu.async_copy(
                    xr.at[pl.ds(base + j * 128, 128)], xb.at[par, j], sem_p))
                ds.append(pltpu.async_copy(
                    yr.at[pl.ds(base + j * 128, 128)], yb.at[par, j], sem_p))
            for d in ds:
                d.wait()

            @pl.loop(0, B // 128)
            def _pass_a(j):
                @pl.loop(0, 8)
                def _grp(gg):
                    o = gg * 16
                    xs = xb[par, j, pl.ds(o, 16)] * 2.0 - 1.0
                    ys = yb[par, j, pl.ds(o, 16)] * 2.0 - 1.0
                    for (r, ix_, _, _, _) in stream_cfg:
                        half = np.float32(0.5 * (r - 1))
                        coff = cid * (r * r)
                        ix0 = ((xs + 1.0) * half).astype(jnp.int32)
                        iy0 = ((ys + 1.0) * half).astype(jnp.int32)
                        iy1 = jnp.minimum(iy0 + 1, r - 1)
                        b0 = iy0 * r + ix0 + coff
                        ix_[par, 0, j, pl.ds(o, 16)] = b0
                        ix_[par, 1, j, pl.ds(o, 16)] = (
                            b0 + (jnp.minimum(iy0 + 1, r - 1) - iy0) * r)

            for j in range(B // 128):
                s = pl.ds(j * 128, 128)
                sem = sems[par]
                for (_, ix_, ra_, rb_, tsh) in stream_cfg:
                    pltpu.async_copy(tsh.at[ix_.at[par, 0, j]],
                                     ra_.at[par, s, :], sem)
                    pltpu.async_copy(tsh.at[ix_.at[par, 1, j]],
                                     rb_.at[par, s, :], sem)
                del tsh

        def drain_rows(par):
            for rows in (r2a, r2b, r3a, r3b, r4a, r4b):
                pltpu.make_async_copy(
                    outr.at[pl.ds(0, B), pl.ds(0, 16)],
                    rows.at[par], sems[par]).wait()

        def blend(par):
            @pl.loop(0, B // 128)
            def _blk(j):
                @pl.loop(0, 8)
                def _grp(gg):
                    o = gg * 16
                    xs = xb[par, j, pl.ds(o, 16)] * 2.0 - 1.0
                    ys = yb[par, j, pl.ds(o, 16)] * 2.0 - 1.0
                    pvec = j * 128 + o + _iota16()
                    for l, r in enumerate(SMALL_RES):
                        rr = r * r
                        ix0, iy0, w00, w01, w10, w11 = _weights(xs, ys, r)
                        ix1 = jnp.minimum(ix0 + 1, r - 1)
                        iy1 = jnp.minimum(iy0 + 1, r - 1)
                        b00 = iy0 * r + ix0
                        dx = ix1 - ix0
                        b01 = b00 + dx
                        b10 = iy1 * r + ix0
                        b11 = b10 + dx
                        for f in range(8):
                            cf = SMALL_BASE[l] + f * rr
                            v00 = plsc.load_gather(st, [b00 + cf])
                            v01 = plsc.load_gather(st, [b01 + cf])
                            v10 = plsc.load_gather(st, [b10 + cf])
                            v11 = plsc.load_gather(st, [b11 + cf])
                            of = (v00 * w00 + v01 * w01
                                  + v10 * w10 + v11 * w11)
                            plsc.store_scatter(
                                ob.at[par],
                                [pvec, jnp.full((16,), l * 8 + f,
                                                jnp.int32)], of)
                    for li, (r, _, ra_, rb_, _) in enumerate(stream_cfg):
                        _, _, w00, w01, w10, w11 = _weights(xs, ys, r)
                        for f in range(8):
                            kf = jnp.full((16,), f, jnp.int32)
                            kf8 = jnp.full((16,), f + 8, jnp.int32)
                            v00 = plsc.load_gather(ra_.at[par], [pvec, kf])
                            v01 = plsc.load_gather(ra_.at[par], [pvec, kf8])
                            v10 = plsc.load_gather(rb_.at[par], [pvec, kf])
                            v11 = plsc.load_gather(rb_.at[par], [pvec, kf8])
                            of = (v00 * w00 + v01 * w01
                                  + v10 * w10 + v11 * w11)
                            plsc.store_scatter(
                                ob.at[par],
                                [pvec, jnp.full((16,), 16 + li * 8 + f,
                                                jnp.int32)], of)

        do_prefetch(0, 0)

        @pl.loop(0, nch, step=2)
        def _chunk2(ch0):
            for sub in range(2):
                ch = ch0 + sub

                @pl.when(ch + 1 < nch)
                def _():
                    do_prefetch(ch + 1, 1 - sub)

                drain_rows(sub)

                @pl.when(ch >= 2)
                def _():
                    pltpu.make_async_copy(
                        outr.at[pl.ds(0, B), :], ob.at[sub], sem_o).wait()

                blend(sub)
                base = wid * pw + ch * B
                pltpu.async_copy(ob.at[sub], outr.at[pl.ds(base, B), :],
                                 sem_o)

        for par in range(2):
            pltpu.make_async_copy(
                outr.at[pl.ds(0, B), :], ob.at[par], sem_o).wait()

    phase2()


def kernel(x, y, F0, F1, F2, F3, F4):
    f32 = jnp.float32
    mesh = plsc.VectorSubcoreMesh(core_axis_name="c", subcore_axis_name="s")
    cparams = pltpu.CompilerParams(needs_layout_passes=False,
                                   use_tc_tiling_on_sc=False)

    main = pl.kernel(
        _body,
        out_type=(
            jax.ShapeDtypeStruct((N_PTS, N_OUT), f32),
            jax.ShapeDtypeStruct((2 * 64 * 64, 16), f32),
            jax.ShapeDtypeStruct((2 * 128 * 128, 16), f32),
            jax.ShapeDtypeStruct((2 * 256 * 256, 16), f32),
        ),
        mesh=mesh,
        scratch_types=[
            pltpu.VMEM_SHARED((ST_SIZE,), f32),
            pltpu.VMEM((2, B // 128, 128), f32),
            pltpu.VMEM((2, B // 128, 128), f32),
            pltpu.VMEM((2, 2, B // 128, 128), jnp.int32),
            pltpu.VMEM((2, 2, B // 128, 128), jnp.int32),
            pltpu.VMEM((2, 2, B // 128, 128), jnp.int32),
            pltpu.VMEM((2, B, 16), f32),
            pltpu.VMEM((2, B, 16), f32),
            pltpu.VMEM((2, B, 16), f32),
            pltpu.VMEM((2, B, 16), f32),
            pltpu.VMEM((2, B, 16), f32),
            pltpu.VMEM((2, B, 16), f32),
            pltpu.VMEM((2, B, N_OUT), f32),
            pltpu.VMEM((ST_SIZE,), f32),
            pltpu.SemaphoreType.DMA,
            pltpu.SemaphoreType.DMA,
            pltpu.SemaphoreType.DMA,
            pltpu.SemaphoreType.DMA,
        ],
        compiler_params=cparams,
    )

    fs = [F.reshape(4, 8, r * r)
          for F, r in zip((F0, F1, F2, F3, F4), (16, 32, 64, 128, 256))]
    return main(x.reshape(-1), y.reshape(-1), *fs)[0]
